# Initial kernel scaffold; baseline (speedup 1.0000x reference)
import jax
import jax.numpy as jnp
from jax.experimental import pallas as pl


def kernel(x, edge_index, W1, b1, W2, b2):
    raise NotImplementedError("write your pallas kernel here")


# speedup vs baseline: 19.7081x; 19.7081x over previous
import functools

import jax
import jax.numpy as jnp
from jax import lax
from jax.experimental import pallas as pl
from jax.experimental.pallas import tpu as pltpu
from jax.experimental.pallas import tpu_sc as plsc

_NC = 2
_NS = 16
_NW = _NC * _NS
_CH = 128


def _fill_vmem(ref, rows, cols, value):
    per_row = cols // 16

    def body(i, _):
        r = i // per_row
        c = (i % per_row) * 16
        ref[r, pl.ds(c, 16)] = jnp.full((16,), value, ref.dtype)
        return 0

    lax.fori_loop(0, rows * per_row, body, 0)


def _fill_vmem1(ref, n, value):
    def body(i, _):
        ref[pl.ds(i * 16, 16)] = jnp.full((16,), value, ref.dtype)
        return 0

    lax.fori_loop(0, n // 16, body, 0)


def _make_deg_kernel(np_, nchunks):
    mesh = plsc.VectorSubcoreMesh(core_axis_name="c", subcore_axis_name="s")
    rpt = np_ // _NS

    @functools.partial(
        pl.kernel,
        mesh=mesh,
        out_type=jax.ShapeDtypeStruct((_NC * np_, _CH), jnp.float32),
        scratch_types=[
            pltpu.VMEM_SHARED((np_, _CH), jnp.float32),
            pltpu.VMEM((nchunks, _CH), jnp.int32),
            pltpu.VMEM((_CH, _CH), jnp.float32),
        ],
    )
    def deg_kernel(dst_hbm, out_hbm, acc, dstv, ones_v):
        c = lax.axis_index("c")
        s = lax.axis_index("s")
        wid = s * _NC + c
        pltpu.sync_copy(dst_hbm.at[wid], dstv)
        _fill_vmem(ones_v, _CH, _CH, 0.0)
        base = s * rpt
        for k in range(rpt // _CH):
            pltpu.sync_copy(ones_v, acc.at[pl.ds(base + k * _CH, _CH)])
        _fill_vmem(ones_v, _CH, _CH, 1.0)
        plsc.subcore_barrier()

        def chunk(j, _):
            pltpu.sync_copy(ones_v, acc.at[dstv.at[j]], add=True)
            return 0

        lax.fori_loop(0, nchunks, chunk, 0)
        plsc.subcore_barrier()
        for k in range(rpt // _CH):
            pltpu.sync_copy(acc.at[pl.ds(base + k * _CH, _CH)], ones_v)
            pltpu.sync_copy(ones_v,
                            out_hbm.at[pl.ds(c * np_ + base + k * _CH, _CH)])

    return deg_kernel


def _make_prop_kernel(np_, nchunks, d):
    mesh = plsc.VectorSubcoreMesh(core_axis_name="c", subcore_axis_name="s")
    rpt = np_ // _NS

    @functools.partial(
        pl.kernel,
        mesh=mesh,
        out_type=jax.ShapeDtypeStruct((_NC * np_, d), jnp.float32),
        scratch_types=[
            pltpu.VMEM_SHARED((np_, d), jnp.float32),
            pltpu.VMEM((nchunks, _CH), jnp.int32),
            pltpu.VMEM((nchunks, _CH), jnp.int32),
            pltpu.VMEM((_CH, d), jnp.float32),
            pltpu.SemaphoreType.DMA,
        ],
    )
    def prop_kernel(z_hbm, src_hbm, dst_hbm, out_hbm, acc, srcv, dstv, rows_v, sem):
        c = lax.axis_index("c")
        s = lax.axis_index("s")
        wid = s * _NC + c
        pltpu.sync_copy(src_hbm.at[wid], srcv)
        pltpu.sync_copy(dst_hbm.at[wid], dstv)
        _fill_vmem(rows_v, _CH, d, 0.0)
        base = s * rpt
        for k in range(rpt // _CH):
            pltpu.sync_copy(rows_v, acc.at[pl.ds(base + k * _CH, _CH)])
        plsc.subcore_barrier()

        def chunk(j, _):
            pltpu.async_copy(z_hbm.at[srcv.at[j]], rows_v, sem).wait()
            pltpu.sync_copy(rows_v, acc.at[dstv.at[j]], add=True)
            return 0

        lax.fori_loop(0, nchunks, chunk, 0)
        plsc.subcore_barrier()
        for k in range(rpt // _CH):
            pltpu.sync_copy(acc.at[pl.ds(base + k * _CH, _CH)], rows_v)
            pltpu.sync_copy(rows_v,
                            out_hbm.at[pl.ds(c * np_ + base + k * _CH, _CH)])

    return prop_kernel


def _dinv_from_deg(degp_ref):
    deg = degp_ref[0, :, 0:1] + degp_ref[1, :, 0:1] + 1.0
    return lax.rsqrt(deg)


def _z1_body(x_ref, w_ref, degp_ref, o_ref):
    dinv = _dinv_from_deg(degp_ref)
    xw = jnp.dot(x_ref[...], w_ref[...], preferred_element_type=jnp.float32)
    o_ref[...] = xw * dinv


def _mid_body(a_ref, z1_ref, degp_ref, b1_ref, w2_ref, o_ref):
    dinv = _dinv_from_deg(degp_ref)
    h = (a_ref[0] + a_ref[1] + z1_ref[...]) * dinv + b1_ref[...]
    h = jnp.maximum(h, 0.0)
    hw = jnp.dot(h, w2_ref[...], preferred_element_type=jnp.float32)
    o_ref[...] = hw * dinv


def _out_body(a_ref, z2_ref, degp_ref, b2_ref, o_ref):
    dinv = _dinv_from_deg(degp_ref)
    dc = o_ref.shape[1]
    s = (a_ref[0] + a_ref[1] + z2_ref[...])[:, :dc]
    o_ref[...] = s * dinv + b2_ref[...]


def kernel(x, edge_index, W1, b1, W2, b2):
    n, dfeat = x.shape
    dh = W1.shape[1]
    dc = W2.shape[1]
    dcp = -(-dc // 128) * 128
    e = edge_index.shape[1]

    np_ = -(-(n + 1) // (_NS * _CH)) * (_NS * _CH)
    ep = -(-e // (_NW * _CH * 8)) * (_NW * _CH * 8)
    nchunks = ep // (_NW * _CH)

    ei = edge_index.astype(jnp.int32)
    pad = n + jnp.arange(ep - e, dtype=jnp.int32) % (np_ - n)
    src = jnp.concatenate([ei[0], pad]).reshape(_NW, nchunks, _CH)
    dst = jnp.concatenate([ei[1], pad]).reshape(_NW, nchunks, _CH)

    xp = jnp.pad(x, ((0, np_ - n), (0, 0)))

    degp = _make_deg_kernel(np_, nchunks)(dst).reshape(_NC, np_, _CH)

    rblk = 2048
    grid = (np_ // rblk,)
    degp_spec = pl.BlockSpec((_NC, rblk, _CH), lambda r: (0, r, 0))

    z1 = pl.pallas_call(
        _z1_body,
        grid=grid,
        in_specs=[
            pl.BlockSpec((rblk, dfeat), lambda r: (r, 0)),
            pl.BlockSpec((dfeat, dh), lambda r: (0, 0)),
            degp_spec,
        ],
        out_specs=pl.BlockSpec((rblk, dh), lambda r: (r, 0)),
        out_shape=jax.ShapeDtypeStruct((np_, dh), jnp.float32),
    )(xp, W1, degp)

    prop1 = _make_prop_kernel(np_, nchunks, dh)
    acc1 = prop1(z1, src, dst).reshape(_NC, np_, dh)

    z2 = pl.pallas_call(
        _mid_body,
        grid=grid,
        in_specs=[
            pl.BlockSpec((_NC, rblk, dh), lambda r: (0, r, 0)),
            pl.BlockSpec((rblk, dh), lambda r: (r, 0)),
            degp_spec,
            pl.BlockSpec((1, dh), lambda r: (0, 0)),
            pl.BlockSpec((dh, dcp), lambda r: (0, 0)),
        ],
        out_specs=pl.BlockSpec((rblk, dcp), lambda r: (r, 0)),
        out_shape=jax.ShapeDtypeStruct((np_, dcp), jnp.float32),
    )(acc1, z1, degp, b1.reshape(1, dh), jnp.pad(W2, ((0, 0), (0, dcp - dc))))

    prop2 = _make_prop_kernel(np_, nchunks, dcp)
    acc2 = prop2(z2, src, dst).reshape(_NC, np_, dcp)

    out = pl.pallas_call(
        _out_body,
        grid=grid,
        in_specs=[
            pl.BlockSpec((_NC, rblk, dcp), lambda r: (0, r, 0)),
            pl.BlockSpec((rblk, dcp), lambda r: (r, 0)),
            degp_spec,
            pl.BlockSpec((1, dc), lambda r: (0, 0)),
        ],
        out_specs=pl.BlockSpec((rblk, dc), lambda r: (r, 0)),
        out_shape=jax.ShapeDtypeStruct((np_, dc), jnp.float32),
    )(acc2, z2, degp, b2.reshape(1, dc))

    return out[:n]
